# baseline (speedup 1.0000x reference)
import functools

import jax
import jax.numpy as jnp
from jax import lax
from jax.experimental import pallas as pl
from jax.experimental.pallas import tpu as pltpu
from jax.experimental.pallas import tpu_sc as plsc

B = 16384
D = 64
NC = 2
NS = 16
NW = NC * NS
BPW = B // NW
CHUNK = 128
NCHUNK = BPW // CHUNK
SPITCH = 17


def _body(uids_hbm, iids_hbm, utab_hbm, itab_hbm, out_hbm,
          uidx_v, iidx_v, urows, irows, part, scores, usem, isem):
    wid = lax.axis_index("s") * NC + lax.axis_index("c")
    base = wid * BPW

    pltpu.sync_copy(uids_hbm.at[wid], uidx_v)
    pltpu.sync_copy(iids_hbm.at[wid], iidx_v)

    copies = []
    for j in range(NCHUNK):
        dst = urows.at[pl.ds(j * CHUNK, CHUNK)]
        copies.append(pltpu.async_copy(utab_hbm.at[uidx_v.at[j]], dst, usem))
    for j in range(NCHUNK):
        dst = irows.at[pl.ds(j * CHUNK, CHUNK)]
        copies.append(pltpu.async_copy(itab_hbm.at[iidx_v.at[j]], dst, isem))
    for c in copies:
        c.wait()

    def row_step(r, carry):
        s = urows[r, pl.ds(0, 16)] * irows[r, pl.ds(0, 16)]
        s += urows[r, pl.ds(16, 16)] * irows[r, pl.ds(16, 16)]
        s += urows[r, pl.ds(32, 16)] * irows[r, pl.ds(32, 16)]
        s += urows[r, pl.ds(48, 16)] * irows[r, pl.ds(48, 16)]
        part[pl.ds(r * SPITCH, 16)] = s
        return carry

    lax.fori_loop(0, BPW, row_step, 0)

    lane = lax.iota(jnp.int32, 16)

    def group_step(g, carry):
        flat = (g * 16 + lane) * SPITCH
        acc = plsc.load_gather(part, [flat])
        for c in range(1, 16):
            acc += plsc.load_gather(part, [flat + c])
        scores[pl.ds(g * 16, 16)] = acc
        return carry

    lax.fori_loop(0, BPW // 16, group_step, 0)

    pltpu.sync_copy(scores, out_hbm.at[pl.ds(base, BPW)])


@functools.partial(
    pl.kernel,
    out_type=jax.ShapeDtypeStruct((B,), jnp.float32),
    mesh=plsc.VectorSubcoreMesh(core_axis_name="c", subcore_axis_name="s"),
    compiler_params=pltpu.CompilerParams(
        needs_layout_passes=False, use_tc_tiling_on_sc=False),
    scratch_types=[
        pltpu.VMEM((NCHUNK, CHUNK), jnp.int32),
        pltpu.VMEM((NCHUNK, CHUNK), jnp.int32),
        pltpu.VMEM((BPW, D), jnp.float32),
        pltpu.VMEM((BPW, D), jnp.float32),
        pltpu.VMEM((BPW * SPITCH,), jnp.float32),
        pltpu.VMEM((BPW,), jnp.float32),
        pltpu.SemaphoreType.DMA,
        pltpu.SemaphoreType.DMA,
    ],
)
def _two_tower_sc(uids_hbm, iids_hbm, utab_hbm, itab_hbm, out_hbm,
                  uidx_v, iidx_v, urows, irows, part, scores, usem, isem):
    _body(uids_hbm, iids_hbm, utab_hbm, itab_hbm, out_hbm,
          uidx_v, iidx_v, urows, irows, part, scores, usem, isem)


@jax.jit
def kernel(user_ids, item_ids, user_table, item_table):
    uids = user_ids.astype(jnp.int32).reshape(NW, NCHUNK, CHUNK)
    iids = item_ids.astype(jnp.int32).reshape(NW, NCHUNK, CHUNK)
    return _two_tower_sc(uids, iids, user_table, item_table)


# speedup vs baseline: 6.3898x; 6.3898x over previous
import functools

import jax
import jax.numpy as jnp
from jax import lax
from jax.experimental import pallas as pl
from jax.experimental.pallas import tpu as pltpu
from jax.experimental.pallas import tpu_sc as plsc

B = 16384
D = 64
NROW = 1000000
NC = 2
NS = 16
NW = NC * NS
BPW = B // NW
CHUNK = 32
NCHUNK = BPW // CHUNK
CSLOT = CHUNK // 8


def _issue_one(tab, ids_v, buf, k, jj, sem):
    r = ids_v[pl.ds(k, 16)][0]
    r8 = pl.multiple_of((r >> 3) << 3, 8)
    col0 = (jj & 7) * 8
    return pltpu.async_copy(
        tab.at[:, :, pl.ds(r8, 8)],
        buf.at[jj >> 3, :, :, pl.ds(col0, 8)],
        sem)


def _body(uids_hbm, iids_hbm, utab_hbm, itab_hbm, out_hbm,
          uids_v, iids_v, ubuf, ibuf, scores, usem, isem):
    wid = lax.axis_index("s") * NC + lax.axis_index("c")
    base = wid * BPW

    pltpu.sync_copy(uids_hbm.at[pl.ds(base, BPW)], uids_v.at[pl.ds(0, BPW)])
    pltpu.sync_copy(iids_hbm.at[pl.ds(base, BPW)], iids_v.at[pl.ds(0, BPW)])

    lane = lax.iota(jnp.int32, 16)

    def chunk_step(ci, carry):
        k0 = ci * CHUNK
        copies = []
        for jj in range(CHUNK):
            copies.append(_issue_one(utab_hbm, uids_v, ubuf, k0 + jj, jj, usem))
            copies.append(_issue_one(itab_hbm, iids_v, ibuf, k0 + jj, jj, isem))
        for c in copies:
            c.wait()

        for g in range(CHUNK // 16):
            kl = g * 16 + lane
            slot_vec = kl >> 3
            rho_u = uids_v[pl.ds(k0 + g * 16, 16)] & 7
            rho_i = iids_v[pl.ds(k0 + g * 16, 16)] & 7
            colu = (kl & 7) * 8 + rho_u
            coli = (kl & 7) * 8 + rho_i
            acc = jnp.zeros((16,), jnp.float32)
            for d in range(D):
                bvec = jnp.full((16,), d >> 3, jnp.int32)
                svec = jnp.full((16,), d & 7, jnp.int32)
                uv = plsc.load_gather(ubuf, [slot_vec, bvec, svec, colu])
                iv = plsc.load_gather(ibuf, [slot_vec, bvec, svec, coli])
                acc += uv * iv
            scores[pl.ds(k0 + g * 16, 16)] = acc
        return carry

    lax.fori_loop(0, NCHUNK, chunk_step, 0)

    pltpu.sync_copy(scores, out_hbm.at[pl.ds(base, BPW)])


@functools.partial(
    pl.kernel,
    out_type=jax.ShapeDtypeStruct((B,), jnp.float32),
    mesh=plsc.VectorSubcoreMesh(core_axis_name="c", subcore_axis_name="s"),
    compiler_params=pltpu.CompilerParams(
        needs_layout_passes=False, use_tc_tiling_on_sc=True),
    scratch_types=[
        pltpu.VMEM((BPW + 16,), jnp.int32),
        pltpu.VMEM((BPW + 16,), jnp.int32),
        pltpu.VMEM((CSLOT, 8, 8, 128), jnp.float32),
        pltpu.VMEM((CSLOT, 8, 8, 128), jnp.float32),
        pltpu.VMEM((BPW,), jnp.float32),
        pltpu.SemaphoreType.DMA,
        pltpu.SemaphoreType.DMA,
    ],
)
def _two_tower_sc(uids_hbm, iids_hbm, utab_hbm, itab_hbm, out_hbm,
                  uids_v, iids_v, ubuf, ibuf, scores, usem, isem):
    _body(uids_hbm, iids_hbm, utab_hbm, itab_hbm, out_hbm,
          uids_v, iids_v, ubuf, ibuf, scores, usem, isem)


@jax.jit
def kernel(user_ids, item_ids, user_table, item_table):
    utabt = user_table.T.reshape(8, 8, NROW)
    itabt = item_table.T.reshape(8, 8, NROW)
    return _two_tower_sc(user_ids.astype(jnp.int32),
                         item_ids.astype(jnp.int32), utabt, itabt)


# speedup vs baseline: 6.8941x; 1.0789x over previous
import functools

import jax
import jax.numpy as jnp
from jax import lax
from jax.experimental import pallas as pl
from jax.experimental.pallas import tpu as pltpu
from jax.experimental.pallas import tpu_sc as plsc

B = 16384
D = 64
NROW = 1000000
NC = 2
NS = 16
NW = NC * NS
BPW = B // NW
CHUNK = 16
NCHUNK = BPW // CHUNK
CSLOT = CHUNK // 8
NPAIR = NCHUNK // 2


def _issue_chunk(tab, ids_v, buf, k0, sem):
    for jj in range(CHUNK):
        r = ids_v[pl.ds(k0 + jj, 16)][0]
        r8 = pl.multiple_of((r >> 3) << 3, 8)
        pltpu.async_copy(
            tab.at[:, :, pl.ds(r8, 8)],
            buf.at[jj >> 3, :, :, pl.ds((jj & 7) * 8, 8)],
            sem)


def _drain_chunk(tab, buf, sem):
    for sl in range(CSLOT):
        pltpu.make_async_copy(tab.at[:, :, pl.ds(0, 64)],
                              buf.at[sl].at[:, :, pl.ds(0, 64)],
                              sem).wait()


def _compute_chunk(uids_v, iids_v, ubuf, ibuf, scores, k0, lane):
    slot_vec = lane >> 3
    rho_u = uids_v[pl.ds(k0, 16)] & 7
    rho_i = iids_v[pl.ds(k0, 16)] & 7
    colu = (lane & 7) * 8 + rho_u
    coli = (lane & 7) * 8 + rho_i
    acc = jnp.zeros((16,), jnp.float32)
    for d in range(D):
        bvec = jnp.full((16,), d >> 3, jnp.int32)
        svec = jnp.full((16,), d & 7, jnp.int32)
        uv = plsc.load_gather(ubuf, [slot_vec, bvec, svec, colu])
        iv = plsc.load_gather(ibuf, [slot_vec, bvec, svec, coli])
        acc += uv * iv
    scores[pl.ds(k0, 16)] = acc


def _body(uids_hbm, iids_hbm, utab_hbm, itab_hbm, out_hbm,
          uids_v, iids_v, ubufA, ibufA, ubufB, ibufB, scores,
          usemA, isemA, usemB, isemB):
    wid = lax.axis_index("s") * NC + lax.axis_index("c")
    base = wid * BPW

    pltpu.sync_copy(uids_hbm.at[pl.ds(base, BPW)], uids_v.at[pl.ds(0, BPW)])
    pltpu.sync_copy(iids_hbm.at[pl.ds(base, BPW)], iids_v.at[pl.ds(0, BPW)])

    lane = lax.iota(jnp.int32, 16)

    _issue_chunk(utab_hbm, uids_v, ubufA, 0, usemA)
    _issue_chunk(itab_hbm, iids_v, ibufA, 0, isemA)

    def pair_step(m, carry):
        k0a = (2 * m) * CHUNK
        k0b = (2 * m + 1) * CHUNK
        _issue_chunk(utab_hbm, uids_v, ubufB, k0b, usemB)
        _issue_chunk(itab_hbm, iids_v, ibufB, k0b, isemB)
        _drain_chunk(utab_hbm, ubufA, usemA)
        _drain_chunk(itab_hbm, ibufA, isemA)
        _compute_chunk(uids_v, iids_v, ubufA, ibufA, scores, k0a, lane)

        @pl.when(m < NPAIR - 1)
        def _():
            k0n = (2 * m + 2) * CHUNK
            _issue_chunk(utab_hbm, uids_v, ubufA, k0n, usemA)
            _issue_chunk(itab_hbm, iids_v, ibufA, k0n, isemA)

        _drain_chunk(utab_hbm, ubufB, usemB)
        _drain_chunk(itab_hbm, ibufB, isemB)
        _compute_chunk(uids_v, iids_v, ubufB, ibufB, scores, k0b, lane)
        return carry

    lax.fori_loop(0, NPAIR, pair_step, 0)

    pltpu.sync_copy(scores, out_hbm.at[pl.ds(base, BPW)])


@functools.partial(
    pl.kernel,
    out_type=jax.ShapeDtypeStruct((B,), jnp.float32),
    mesh=plsc.VectorSubcoreMesh(core_axis_name="c", subcore_axis_name="s"),
    compiler_params=pltpu.CompilerParams(
        needs_layout_passes=False, use_tc_tiling_on_sc=True),
    scratch_types=[
        pltpu.VMEM((BPW + 16,), jnp.int32),
        pltpu.VMEM((BPW + 16,), jnp.int32),
        pltpu.VMEM((CSLOT, 8, 8, 128), jnp.float32),
        pltpu.VMEM((CSLOT, 8, 8, 128), jnp.float32),
        pltpu.VMEM((CSLOT, 8, 8, 128), jnp.float32),
        pltpu.VMEM((CSLOT, 8, 8, 128), jnp.float32),
        pltpu.VMEM((BPW,), jnp.float32),
        pltpu.SemaphoreType.DMA,
        pltpu.SemaphoreType.DMA,
        pltpu.SemaphoreType.DMA,
        pltpu.SemaphoreType.DMA,
    ],
)
def _two_tower_sc(uids_hbm, iids_hbm, utab_hbm, itab_hbm, out_hbm,
                  uids_v, iids_v, ubufA, ibufA, ubufB, ibufB, scores,
                  usemA, isemA, usemB, isemB):
    _body(uids_hbm, iids_hbm, utab_hbm, itab_hbm, out_hbm,
          uids_v, iids_v, ubufA, ibufA, ubufB, ibufB, scores,
          usemA, isemA, usemB, isemB)


@jax.jit
def kernel(user_ids, item_ids, user_table, item_table):
    utabt = user_table.T.reshape(8, 8, NROW)
    itabt = item_table.T.reshape(8, 8, NROW)
    return _two_tower_sc(user_ids.astype(jnp.int32),
                         item_ids.astype(jnp.int32), utabt, itabt)


# speedup vs baseline: 7.2900x; 1.0574x over previous
import functools

import jax
import jax.numpy as jnp
from jax import lax
from jax.experimental import pallas as pl
from jax.experimental.pallas import tpu as pltpu
from jax.experimental.pallas import tpu_sc as plsc

B = 16384
D = 64
NROW = 1000000
NC = 2
NS = 16
NW = NC * NS
BPW = B // NW
CHUNK = 16
NCHUNK = BPW // CHUNK
CSLOT = CHUNK // 8
NPAIR = NCHUNK // 2


def _issue_chunk(tab, ids_v, buf, k0, sem):
    r8v = (ids_v[pl.ds(k0, 16)] >> 3) << 3
    for jj in range(CHUNK):
        r8 = pl.multiple_of(r8v[jj], 8)
        pltpu.async_copy(
            tab.at[:, :, pl.ds(r8, 8)],
            buf.at[jj >> 3, :, :, pl.ds((jj & 7) * 8, 8)],
            sem)


def _drain_chunk(tab, buf, sem):
    for sl in range(CSLOT):
        pltpu.make_async_copy(tab.at[:, :, pl.ds(0, 64)],
                              buf.at[sl].at[:, :, pl.ds(0, 64)],
                              sem).wait()


def _compute_chunk(uids_v, iids_v, ubuf, ibuf, scores, k0, lane):
    slot_vec = lane >> 3
    rho_u = uids_v[pl.ds(k0, 16)] & 7
    rho_i = iids_v[pl.ds(k0, 16)] & 7
    colu = (lane & 7) * 8 + rho_u
    coli = (lane & 7) * 8 + rho_i
    acc = jnp.zeros((16,), jnp.float32)
    for d in range(D):
        bvec = jnp.full((16,), d >> 3, jnp.int32)
        svec = jnp.full((16,), d & 7, jnp.int32)
        uv = plsc.load_gather(ubuf, [slot_vec, bvec, svec, colu])
        iv = plsc.load_gather(ibuf, [slot_vec, bvec, svec, coli])
        acc += uv * iv
    scores[pl.ds(k0, 16)] = acc


def _body(uids_hbm, iids_hbm, utab_hbm, itab_hbm, out_hbm,
          uids_v, iids_v, ubufA, ibufA, ubufB, ibufB, scores,
          usemA, isemA, usemB, isemB):
    wid = lax.axis_index("s") * NC + lax.axis_index("c")
    base = wid * BPW

    pltpu.sync_copy(uids_hbm.at[pl.ds(base, BPW)], uids_v.at[pl.ds(0, BPW)])
    pltpu.sync_copy(iids_hbm.at[pl.ds(base, BPW)], iids_v.at[pl.ds(0, BPW)])

    lane = lax.iota(jnp.int32, 16)

    _issue_chunk(utab_hbm, uids_v, ubufA, 0, usemA)
    _issue_chunk(itab_hbm, iids_v, ibufA, 0, isemA)

    def pair_step(m, carry):
        k0a = (2 * m) * CHUNK
        k0b = (2 * m + 1) * CHUNK
        _issue_chunk(utab_hbm, uids_v, ubufB, k0b, usemB)
        _issue_chunk(itab_hbm, iids_v, ibufB, k0b, isemB)
        _drain_chunk(utab_hbm, ubufA, usemA)
        _drain_chunk(itab_hbm, ibufA, isemA)
        _compute_chunk(uids_v, iids_v, ubufA, ibufA, scores, k0a, lane)

        @pl.when(m < NPAIR - 1)
        def _():
            k0n = (2 * m + 2) * CHUNK
            _issue_chunk(utab_hbm, uids_v, ubufA, k0n, usemA)
            _issue_chunk(itab_hbm, iids_v, ibufA, k0n, isemA)

        _drain_chunk(utab_hbm, ubufB, usemB)
        _drain_chunk(itab_hbm, ibufB, isemB)
        _compute_chunk(uids_v, iids_v, ubufB, ibufB, scores, k0b, lane)
        return carry

    lax.fori_loop(0, NPAIR, pair_step, 0)

    pltpu.sync_copy(scores, out_hbm.at[pl.ds(base, BPW)])


@functools.partial(
    pl.kernel,
    out_type=jax.ShapeDtypeStruct((B,), jnp.float32),
    mesh=plsc.VectorSubcoreMesh(core_axis_name="c", subcore_axis_name="s"),
    compiler_params=pltpu.CompilerParams(
        needs_layout_passes=False, use_tc_tiling_on_sc=True),
    scratch_types=[
        pltpu.VMEM((BPW + 16,), jnp.int32),
        pltpu.VMEM((BPW + 16,), jnp.int32),
        pltpu.VMEM((CSLOT, 8, 8, 128), jnp.float32),
        pltpu.VMEM((CSLOT, 8, 8, 128), jnp.float32),
        pltpu.VMEM((CSLOT, 8, 8, 128), jnp.float32),
        pltpu.VMEM((CSLOT, 8, 8, 128), jnp.float32),
        pltpu.VMEM((BPW,), jnp.float32),
        pltpu.SemaphoreType.DMA,
        pltpu.SemaphoreType.DMA,
        pltpu.SemaphoreType.DMA,
        pltpu.SemaphoreType.DMA,
    ],
)
def _two_tower_sc(uids_hbm, iids_hbm, utab_hbm, itab_hbm, out_hbm,
                  uids_v, iids_v, ubufA, ibufA, ubufB, ibufB, scores,
                  usemA, isemA, usemB, isemB):
    _body(uids_hbm, iids_hbm, utab_hbm, itab_hbm, out_hbm,
          uids_v, iids_v, ubufA, ibufA, ubufB, ibufB, scores,
          usemA, isemA, usemB, isemB)


@jax.jit
def kernel(user_ids, item_ids, user_table, item_table):
    utabt = user_table.T.reshape(8, 8, NROW)
    itabt = item_table.T.reshape(8, 8, NROW)
    return _two_tower_sc(user_ids.astype(jnp.int32),
                         item_ids.astype(jnp.int32), utabt, itabt)


# speedup vs baseline: 8.2607x; 1.1332x over previous
import functools

import jax
import jax.numpy as jnp
from jax import lax
from jax.experimental import pallas as pl
from jax.experimental.pallas import tpu as pltpu
from jax.experimental.pallas import tpu_sc as plsc

B = 16384
D = 64
NROW = 1000000
NC = 2
NS = 16
NW = NC * NS
BPW = B // NW
CHUNK = 32
NCHUNK = BPW // CHUNK
CSLOT = CHUNK // 16
NPAIR = NCHUNK // 2


def _issue_chunk(tab, ids_v, buf, k0, sem):
    def issue_group(g, carry):
        r8v = (ids_v[pl.ds(k0 + g * 16, 16)] >> 3) << 3
        for jl in range(16):
            r8 = pl.multiple_of(r8v[jl], 8)
            pltpu.async_copy(
                tab.at[:, :, pl.ds(r8, 8)],
                buf.at[g, :, :, pl.ds(jl * 8, 8)],
                sem)
        return carry

    lax.fori_loop(0, CHUNK // 16, issue_group, 0)


def _drain_chunk(tab, buf, sem):
    for sl in range(CSLOT):
        pltpu.make_async_copy(tab.at[:, :, pl.ds(0, 128)], buf.at[sl],
                              sem).wait()


def _compute_chunk(uids_v, iids_v, ubuf, ibuf, scores, k0, lane):
    for g in range(CHUNK // 16):
        slot_vec = jnp.full((16,), g, jnp.int32)
        rho_u = uids_v[pl.ds(k0 + g * 16, 16)] & 7
        rho_i = iids_v[pl.ds(k0 + g * 16, 16)] & 7
        colu = lane * 8 + rho_u
        coli = lane * 8 + rho_i
        acc = jnp.zeros((16,), jnp.float32)
        for d in range(D):
            bvec = jnp.full((16,), d >> 3, jnp.int32)
            svec = jnp.full((16,), d & 7, jnp.int32)
            uv = plsc.load_gather(ubuf, [slot_vec, bvec, svec, colu])
            iv = plsc.load_gather(ibuf, [slot_vec, bvec, svec, coli])
            acc += uv * iv
        scores[pl.ds(k0 + g * 16, 16)] = acc


def _body(uids_hbm, iids_hbm, utab_hbm, itab_hbm, out_hbm,
          uids_v, iids_v, ubufA, ibufA, ubufB, ibufB, scores,
          usemA, isemA, usemB, isemB):
    wid = lax.axis_index("s") * NC + lax.axis_index("c")
    base = wid * BPW

    pltpu.sync_copy(uids_hbm.at[pl.ds(base, BPW)], uids_v.at[pl.ds(0, BPW)])
    pltpu.sync_copy(iids_hbm.at[pl.ds(base, BPW)], iids_v.at[pl.ds(0, BPW)])

    lane = lax.iota(jnp.int32, 16)

    _issue_chunk(utab_hbm, uids_v, ubufA, 0, usemA)
    _issue_chunk(itab_hbm, iids_v, ibufA, 0, isemA)

    def pair_step(m, carry):
        k0a = (2 * m) * CHUNK
        k0b = (2 * m + 1) * CHUNK
        _issue_chunk(utab_hbm, uids_v, ubufB, k0b, usemB)
        _issue_chunk(itab_hbm, iids_v, ibufB, k0b, isemB)
        _drain_chunk(utab_hbm, ubufA, usemA)
        _drain_chunk(itab_hbm, ibufA, isemA)
        _compute_chunk(uids_v, iids_v, ubufA, ibufA, scores, k0a, lane)

        @pl.when(m < NPAIR - 1)
        def _():
            k0n = (2 * m + 2) * CHUNK
            _issue_chunk(utab_hbm, uids_v, ubufA, k0n, usemA)
            _issue_chunk(itab_hbm, iids_v, ibufA, k0n, isemA)

        _drain_chunk(utab_hbm, ubufB, usemB)
        _drain_chunk(itab_hbm, ibufB, isemB)
        _compute_chunk(uids_v, iids_v, ubufB, ibufB, scores, k0b, lane)
        return carry

    lax.fori_loop(0, NPAIR, pair_step, 0)

    pltpu.sync_copy(scores, out_hbm.at[pl.ds(base, BPW)])


@functools.partial(
    pl.kernel,
    out_type=jax.ShapeDtypeStruct((B,), jnp.float32),
    mesh=plsc.VectorSubcoreMesh(core_axis_name="c", subcore_axis_name="s"),
    compiler_params=pltpu.CompilerParams(
        needs_layout_passes=False, use_tc_tiling_on_sc=True),
    scratch_types=[
        pltpu.VMEM((BPW + 16,), jnp.int32),
        pltpu.VMEM((BPW + 16,), jnp.int32),
        pltpu.VMEM((CSLOT, 8, 8, 128), jnp.float32),
        pltpu.VMEM((CSLOT, 8, 8, 128), jnp.float32),
        pltpu.VMEM((CSLOT, 8, 8, 128), jnp.float32),
        pltpu.VMEM((CSLOT, 8, 8, 128), jnp.float32),
        pltpu.VMEM((BPW,), jnp.float32),
        pltpu.SemaphoreType.DMA,
        pltpu.SemaphoreType.DMA,
        pltpu.SemaphoreType.DMA,
        pltpu.SemaphoreType.DMA,
    ],
)
def _two_tower_sc(uids_hbm, iids_hbm, utab_hbm, itab_hbm, out_hbm,
                  uids_v, iids_v, ubufA, ibufA, ubufB, ibufB, scores,
                  usemA, isemA, usemB, isemB):
    _body(uids_hbm, iids_hbm, utab_hbm, itab_hbm, out_hbm,
          uids_v, iids_v, ubufA, ibufA, ubufB, ibufB, scores,
          usemA, isemA, usemB, isemB)


@jax.jit
def kernel(user_ids, item_ids, user_table, item_table):
    utabt = user_table.T.reshape(8, 8, NROW)
    itabt = item_table.T.reshape(8, 8, NROW)
    return _two_tower_sc(user_ids.astype(jnp.int32),
                         item_ids.astype(jnp.int32), utabt, itabt)
